# baseline (speedup 1.0000x reference)
import dataclasses
import functools

import jax
import jax.numpy as jnp
from jax import lax
from jax.experimental import pallas as pl
from jax.experimental.pallas import tpu as pltpu
from jax.experimental.pallas import tpu_sc as plsc

B = 16384
D = 64
NC = 2
NS = 16
NW = NC * NS
BPW = B // NW
CHI = 128
NCH = BPW // CHI
L = 16


def _rec_mf_body(users_hbm, items_hbm, ut_hbm, it_hbm, out_hbm,
                 uidx, iidx, urows, irows, outv, sem):
    wid = lax.axis_index("s") * NC + lax.axis_index("c")
    base = wid * BPW

    pltpu.sync_copy(users_hbm.at[pl.ds(wid * NCH, NCH)], uidx)
    pltpu.sync_copy(items_hbm.at[pl.ds(wid * NCH, NCH)], iidx)

    copies = []
    for j in range(NCH):
        copies.append(pltpu.async_copy(
            ut_hbm.at[uidx.at[j]], urows.at[pl.ds(j * CHI, CHI)], sem))
        copies.append(pltpu.async_copy(
            it_hbm.at[iidx.at[j]], irows.at[pl.ds(j * CHI, CHI)], sem))
    for c in copies:
        c.wait()

    lane_iota = lax.broadcasted_iota(jnp.int32, (L,), 0)

    @pl.loop(0, BPW, step=L)
    def _(g):
        resv = jnp.zeros((L,), jnp.float32)
        for k in range(L):
            r = g + k
            acc = urows[r, pl.ds(0, L)] * irows[r, pl.ds(0, L)]
            for c in range(1, D // L):
                acc = acc + (urows[r, pl.ds(c * L, L)]
                             * irows[r, pl.ds(c * L, L)])
            resv = jnp.where(lane_iota == k, jnp.sum(acc), resv)
        outv[pl.ds(g, L)] = 1.0 / (1.0 + jnp.exp(-resv))

    pltpu.sync_copy(outv, out_hbm.at[pl.ds(base, BPW)])


@jax.jit
def kernel(users, items, user_table, item_table):
    mesh = plsc.VectorSubcoreMesh(core_axis_name="c", subcore_axis_name="s")
    cp = pltpu.CompilerParams(use_tc_tiling_on_sc=False)
    if "needs_layout_passes" in pltpu.CompilerParams.__dataclass_fields__:
        cp = dataclasses.replace(cp, needs_layout_passes=False)
    k = pl.kernel(
        _rec_mf_body,
        out_type=jax.ShapeDtypeStruct((B,), jnp.float32),
        mesh=mesh,
        compiler_params=cp,
        scratch_types=[
            pltpu.VMEM((NCH, CHI), jnp.int32),
            pltpu.VMEM((NCH, CHI), jnp.int32),
            pltpu.VMEM((BPW, D), jnp.float32),
            pltpu.VMEM((BPW, D), jnp.float32),
            pltpu.VMEM((BPW,), jnp.float32),
            pltpu.SemaphoreType.DMA,
        ],
    )
    users2d = users.reshape(NW * NCH, CHI).astype(jnp.int32)
    items2d = items.reshape(NW * NCH, CHI).astype(jnp.int32)
    return k(users2d, items2d, user_table, item_table)


# speedup vs baseline: 1.5672x; 1.5672x over previous
import dataclasses
import functools

import jax
import jax.numpy as jnp
from jax import lax
from jax.experimental import pallas as pl
from jax.experimental.pallas import tpu as pltpu
from jax.experimental.pallas import tpu_sc as plsc

B = 16384
D = 64
NC = 2
NS = 16
NW = NC * NS
BPW = B // NW
CH = 256
L = 16


def _rec_mf_body(users_hbm, items_hbm, ut_hbm, it_hbm, out_hbm,
                 uidx, iidx, urows, irows, outv, sem):
    wid = lax.axis_index("s") * NC + lax.axis_index("c")
    base = wid * BPW

    pltpu.sync_copy(users_hbm.at[pl.ds(base, BPW)], uidx)
    pltpu.sync_copy(items_hbm.at[pl.ds(base, BPW)], iidx)

    lane_iota = lax.broadcasted_iota(jnp.int32, (L,), 0)

    for ch in range(BPW // CH):
        off = ch * CH

        @pl.loop(0, CH, step=L)
        def _(g):
            uvec = uidx[pl.ds(off + g, L)]
            ivec = iidx[pl.ds(off + g, L)]
            for k in range(L):
                pltpu.async_copy(ut_hbm.at[pl.ds(uvec[k], 1)],
                                 urows.at[pl.ds(g + k, 1)], sem)
                pltpu.async_copy(it_hbm.at[pl.ds(ivec[k], 1)],
                                 irows.at[pl.ds(g + k, 1)], sem)

        pltpu.make_async_copy(ut_hbm.at[pl.ds(0, CH)], urows, sem).wait()
        pltpu.make_async_copy(it_hbm.at[pl.ds(0, CH)], irows, sem).wait()

        @pl.loop(0, CH, step=L)
        def _(g):
            resv = jnp.zeros((L,), jnp.float32)
            for k in range(L):
                r = g + k
                acc = urows[r, pl.ds(0, L)] * irows[r, pl.ds(0, L)]
                for c in range(1, D // L):
                    acc = acc + (urows[r, pl.ds(c * L, L)]
                                 * irows[r, pl.ds(c * L, L)])
                resv = jnp.where(lane_iota == k, jnp.sum(acc), resv)
            outv[pl.ds(off + g, L)] = 1.0 / (1.0 + jnp.exp(-resv))

    pltpu.sync_copy(outv, out_hbm.at[pl.ds(base, BPW)])


@jax.jit
def kernel(users, items, user_table, item_table):
    mesh = plsc.VectorSubcoreMesh(core_axis_name="c", subcore_axis_name="s")
    cp = pltpu.CompilerParams()
    if "needs_layout_passes" in pltpu.CompilerParams.__dataclass_fields__:
        cp = dataclasses.replace(cp, needs_layout_passes=False)
    k = pl.kernel(
        _rec_mf_body,
        out_type=jax.ShapeDtypeStruct((B,), jnp.float32),
        mesh=mesh,
        compiler_params=cp,
        scratch_types=[
            pltpu.VMEM((BPW,), jnp.int32),
            pltpu.VMEM((BPW,), jnp.int32),
            pltpu.VMEM((CH, D), jnp.float32),
            pltpu.VMEM((CH, D), jnp.float32),
            pltpu.VMEM((BPW,), jnp.float32),
            pltpu.SemaphoreType.DMA,
        ],
    )
    return k(users.astype(jnp.int32), items.astype(jnp.int32),
             user_table, item_table)
